# Initial kernel scaffold; baseline (speedup 1.0000x reference)
import jax
import jax.numpy as jnp
from jax.experimental import pallas as pl


def kernel(md_1_for, means2D):
    raise NotImplementedError("write your pallas kernel here")


# speedup vs baseline: 4.1504x; 4.1504x over previous
import functools

import jax
import jax.numpy as jnp
from jax import lax
from jax.experimental import pallas as pl
from jax.experimental.pallas import tpu as pltpu
from jax.experimental.pallas import tpu_sc as plsc

H, W = 360, 480
NPIX = H * W
N = 100000
NP = 102400
ROWS = NP // 128
TROWS = ROWS // 16
CB = 10
NCHUNK = TROWS // CB
PIX_W = NPIX // 32
PIX_C = NPIX // 16
CF = 1080
CFP = CF + 8
NVF = (CF + 15) // 16
STG = 3 * CF + 24


def _pre_body(c0, c1, c2, op, dp, mx, my, o0, o1, o2, o3, o4, idx_ref):
    w = 1.0 / (1.0 + jnp.exp(-op[...]))
    x = dp[...]
    d = jnp.maximum(x, 0.0) + jnp.log(1.0 + jnp.exp(-jnp.abs(x)))
    rid = lax.broadcasted_iota(jnp.int32, (ROWS, 128), 0)
    cid = lax.broadcasted_iota(jnp.int32, (ROWS, 128), 1)
    valid = (rid * 128 + cid) < N
    w = jnp.where(valid, w, 0.0)
    px = jnp.clip(jnp.floor(mx[...] * W), 0.0, W - 1)
    py = jnp.clip(jnp.floor(my[...] * H), 0.0, H - 1)
    flat = (py * W + px).astype(jnp.int32)
    idx_ref[...] = jnp.where(valid, flat, rid * 128 + cid)
    o0[...] = c0[...] * w
    o1[...] = c1[...] * w
    o2[...] = c2[...] * w
    o3[...] = d * w
    o4[...] = w


_pre = pl.pallas_call(
    _pre_body,
    out_shape=(
        [jax.ShapeDtypeStruct((ROWS, 128), jnp.float32) for _ in range(5)]
        + [jax.ShapeDtypeStruct((ROWS, 128), jnp.int32)]
    ),
)


@functools.cache
def _make_sc():
  return pl.kernel(
    _sc_body,
    mesh=plsc.VectorSubcoreMesh(core_axis_name="c", subcore_axis_name="s",
                                num_cores=2, num_subcores=16),
    out_type=[
        jax.ShapeDtypeStruct((NPIX * 3,), jnp.float32),
        jax.ShapeDtypeStruct((NPIX,), jnp.float32),
        jax.ShapeDtypeStruct((NPIX,), jnp.float32),
    ],
    scratch_types=[
        pltpu.VMEM((TROWS, 128), jnp.int32),
        pltpu.VMEM((CB, 128), jnp.float32),
        pltpu.VMEM((CB, 128), jnp.float32),
        pltpu.VMEM((CB, 128), jnp.float32),
        pltpu.VMEM((CB, 128), jnp.float32),
        pltpu.VMEM((CB, 128), jnp.float32),
        pltpu.VMEM((CFP,), jnp.float32),
        pltpu.VMEM((CFP,), jnp.float32),
        pltpu.VMEM((CFP,), jnp.float32),
        pltpu.VMEM((CFP,), jnp.float32),
        pltpu.VMEM((CFP,), jnp.float32),
        pltpu.VMEM((STG,), jnp.float32),
        pltpu.VMEM_SHARED((NPIX,), jnp.float32),
        pltpu.VMEM_SHARED((NPIX,), jnp.float32),
        pltpu.VMEM_SHARED((NPIX,), jnp.float32),
        pltpu.VMEM_SHARED((NPIX,), jnp.float32),
        pltpu.VMEM_SHARED((NPIX,), jnp.float32),
        pltpu.SemaphoreType.DMA,
    ],
    compiler_params=pltpu.CompilerParams(needs_layout_passes=False),
  )


def _sc_body(v0h, v1h, v2h, v3h, v4h, idx_hbm, outc, outd, outs,
             idxv, v0, v1, v2, v3, v4, f0, f1, f2, f3, f4, stage,
             a0, a1, a2, a3, a4, sem):
    cc = lax.axis_index("c")
    ss = lax.axis_index("s")
    wid = cc * 16 + ss
    vbufs = (v0, v1, v2, v3, v4)
    fbufs = (f0, f1, f2, f3, f4)
    accs = (a0, a1, a2, a3, a4)
    vhbm = (v0h, v1h, v2h, v3h, v4h)

    def _zb(i, carry):
        stage[pl.ds(i * 16, 16)] = jnp.zeros((16,), jnp.float32)
        return carry
    lax.fori_loop(0, STG // 16, _zb, 0)
    zlen = 3 * CF
    for a in accs:
        for k in range(PIX_C // zlen):
            pltpu.sync_copy(stage.at[pl.ds(0, zlen)],
                            a.at[pl.ds(ss * PIX_C + k * zlen, zlen)])
        rem = PIX_C - (PIX_C // zlen) * zlen
        pltpu.sync_copy(stage.at[pl.ds(0, rem)],
                        a.at[pl.ds(ss * PIX_C + PIX_C - rem, rem)])
    plsc.subcore_barrier()

    pltpu.sync_copy(idx_hbm.at[ss], idxv)
    for c in range(NCHUNK):
        for ch in range(5):
            pltpu.sync_copy(vhbm[ch].at[ss, c], vbufs[ch])

        def _scat(j, carry):
            copies = [
                pltpu.async_copy(vbufs[ch].at[j],
                                 accs[ch].at[idxv.at[c * CB + j]], sem,
                                 add=True)
                for ch in range(5)
            ]
            for cp in copies:
                cp.wait()
            return carry
        lax.fori_loop(0, CB, _scat, 0)
    plsc.subcore_barrier()

    lane3 = lax.iota(jnp.int32, 16) * 3
    for k in range(PIX_W // CF):
        base = wid * PIX_W + k * CF
        for ch in range(5):
            pltpu.sync_copy(accs[ch].at[pl.ds(base, CF)],
                            fbufs[ch].at[pl.ds(0, CF)])

        def _fb(i, carry):
            o = pl.ds(i * 16, 16)
            wv = f4[o]
            inv = 1.0 / jnp.maximum(wv, 1e-8)
            sidx = lane3 + i * 48
            plsc.store_scatter(stage, [sidx], f0[o] * inv)
            plsc.store_scatter(stage, [sidx + 1], f1[o] * inv)
            plsc.store_scatter(stage, [sidx + 2], f2[o] * inv)
            f3[o] = f3[o] * inv
            f4[o] = jnp.clip(wv, 0.0, 1.0)
            return carry
        lax.fori_loop(0, NVF, _fb, 0)

        pltpu.sync_copy(stage.at[pl.ds(0, 3 * CF)],
                        outc.at[pl.ds(base * 3, 3 * CF)])
        pltpu.sync_copy(f3.at[pl.ds(0, CF)], outd.at[pl.ds(base, CF)])
        pltpu.sync_copy(f4.at[pl.ds(0, CF)], outs.at[pl.ds(base, CF)])


def kernel(md_1_for, means2D):
    mdp = jnp.pad(md_1_for, ((0, NP - N), (0, 0)))
    mep = jnp.pad(means2D, ((0, NP - N), (0, 0)))
    cols = [mdp[:, i].reshape(ROWS, 128) for i in (0, 1, 2, 3, 4)]
    mx = mep[:, 0].reshape(ROWS, 128)
    my = mep[:, 1].reshape(ROWS, 128)
    o0, o1, o2, o3, o4, idx = _pre(cols[0], cols[1], cols[2], cols[3],
                                   cols[4], mx, my)
    r4 = lambda a: a.reshape(16, NCHUNK, CB, 128)
    outc, outd, outs = _make_sc()(r4(o0), r4(o1), r4(o2), r4(o3), r4(o4),
                                  idx.reshape(16, TROWS, 128))
    return (outc.reshape(H, W, 3),
            outd.reshape(H, W, 1),
            outs.reshape(H, W, 1))


# speedup vs baseline: 4.6959x; 1.1314x over previous
import functools

import jax
import jax.numpy as jnp
from jax import lax
from jax.experimental import pallas as pl
from jax.experimental.pallas import tpu as pltpu
from jax.experimental.pallas import tpu_sc as plsc

H, W = 360, 480
NPIX = H * W
N = 100000
NP = 102400
ROWS = NP // 128
TROWS = ROWS // 16
CB = 10
CPTS = CB * 128
NCHUNK = TROWS // CB
PIX_W = NPIX // 32
PIX_C = NPIX // 16
CF = 1080
CFP = CF + 8
NVF = (CF + 15) // 16
STG = 3 * CF + 24


def _pre_body(c0, c1, c2, op, dp, mx, my, o0, o1, o2, o3, o4, idx_ref):
    w = 1.0 / (1.0 + jnp.exp(-op[...]))
    x = dp[...]
    d = jnp.maximum(x, 0.0) + jnp.log(1.0 + jnp.exp(-jnp.abs(x)))
    rid = lax.broadcasted_iota(jnp.int32, (ROWS, 128), 0)
    cid = lax.broadcasted_iota(jnp.int32, (ROWS, 128), 1)
    valid = (rid * 128 + cid) < N
    w = jnp.where(valid, w, 0.0)
    px = jnp.clip(jnp.floor(mx[...] * W), 0.0, W - 1)
    py = jnp.clip(jnp.floor(my[...] * H), 0.0, H - 1)
    flat = (py * W + px).astype(jnp.int32)
    idx_ref[...] = jnp.where(valid, flat, rid * 128 + cid)
    o0[...] = c0[...] * w
    o1[...] = c1[...] * w
    o2[...] = c2[...] * w
    o3[...] = d * w
    o4[...] = w


_pre = pl.pallas_call(
    _pre_body,
    out_shape=(
        [jax.ShapeDtypeStruct((ROWS, 128), jnp.float32) for _ in range(5)]
        + [jax.ShapeDtypeStruct((ROWS, 128), jnp.int32)]
    ),
)


@functools.cache
def _make_sc():
  return pl.kernel(
    _sc_body,
    mesh=plsc.VectorSubcoreMesh(core_axis_name="c", subcore_axis_name="s",
                                num_cores=2, num_subcores=16),
    out_type=[
        jax.ShapeDtypeStruct((NPIX * 3,), jnp.float32),
        jax.ShapeDtypeStruct((NPIX,), jnp.float32),
        jax.ShapeDtypeStruct((NPIX,), jnp.float32),
    ],
    scratch_types=[
        pltpu.VMEM((CPTS,), jnp.int32),
        [pltpu.VMEM((CPTS,), jnp.float32) for _ in range(5)],
        [pltpu.VMEM((CPTS,), jnp.float32) for _ in range(5)],
        [pltpu.VMEM((CFP,), jnp.float32) for _ in range(5)],
        pltpu.VMEM((STG,), jnp.float32),
        [pltpu.VMEM_SHARED((NPIX,), jnp.float32) for _ in range(5)],
        pltpu.SemaphoreType.DMA,
        pltpu.SemaphoreType.DMA,
    ],
    compiler_params=pltpu.CompilerParams(needs_layout_passes=False),
  )


def _sc_body(v0h, v1h, v2h, v3h, v4h, idx_hbm, outc, outd, outs,
             idxv, vbufA, vbufB, fbufs, stage, accs, sem_s, sem_l):
    cc = lax.axis_index("c")
    ss = lax.axis_index("s")
    wid = cc * 16 + ss
    vhbm = (v0h, v1h, v2h, v3h, v4h)
    vbufs = (vbufA, vbufB)

    def _zb(i, carry):
        stage[pl.ds(i * 16, 16)] = jnp.zeros((16,), jnp.float32)
        return carry
    lax.fori_loop(0, STG // 16, _zb, 0)
    zlen = 3 * CF
    zcopies = []
    for a in accs:
        for k in range(PIX_C // zlen):
            zcopies.append(pltpu.async_copy(
                stage.at[pl.ds(0, zlen)],
                a.at[pl.ds(ss * PIX_C + k * zlen, zlen)], sem_l))
        rem = PIX_C - (PIX_C // zlen) * zlen
        zcopies.append(pltpu.async_copy(
            stage.at[pl.ds(0, rem)],
            a.at[pl.ds(ss * PIX_C + PIX_C - rem, rem)], sem_l))
    for cp in zcopies:
        cp.wait()
    plsc.subcore_barrier()

    base_pt = ss * TROWS * 128

    def _load(c, buf, sem):
        return [pltpu.async_copy(
            vhbm[ch].at[pl.ds(base_pt + c * CPTS, CPTS)], buf[ch], sem)
            for ch in range(5)]

    pltpu.sync_copy(idx_hbm.at[pl.ds(base_pt, CPTS)], idxv)
    for cp in _load(0, vbufs[0], sem_l):
        cp.wait()
    for c in range(NCHUNK):
        cur = vbufs[c % 2]
        streams = [pltpu.async_copy(cur[ch], accs[ch].at[idxv], sem_s,
                                    add=True) for ch in range(5)]
        loads = _load(c + 1, vbufs[(c + 1) % 2], sem_l) if c + 1 < NCHUNK \
            else []
        for cp in loads:
            cp.wait()
        for cp in streams:
            cp.wait()
        if c + 1 < NCHUNK:
            pltpu.sync_copy(idx_hbm.at[pl.ds(base_pt + (c + 1) * CPTS, CPTS)],
                            idxv)
    plsc.subcore_barrier()

    f0, f1, f2, f3, f4 = fbufs
    lane3 = lax.iota(jnp.int32, 16) * 3
    for k in range(PIX_W // CF):
        base = wid * PIX_W + k * CF
        floads = [pltpu.async_copy(accs[ch].at[pl.ds(base, CF)],
                                   fbufs[ch].at[pl.ds(0, CF)], sem_l)
                  for ch in range(5)]
        for cp in floads:
            cp.wait()

        def _fb(i, carry):
            o = pl.ds(i * 16, 16)
            wv = f4[o]
            inv = 1.0 / jnp.maximum(wv, 1e-8)
            sidx = lane3 + i * 48
            plsc.store_scatter(stage, [sidx], f0[o] * inv)
            plsc.store_scatter(stage, [sidx + 1], f1[o] * inv)
            plsc.store_scatter(stage, [sidx + 2], f2[o] * inv)
            f3[o] = f3[o] * inv
            f4[o] = jnp.clip(wv, 0.0, 1.0)
            return carry
        lax.fori_loop(0, NVF, _fb, 0)

        pltpu.sync_copy(stage.at[pl.ds(0, 3 * CF)],
                        outc.at[pl.ds(base * 3, 3 * CF)])
        pltpu.sync_copy(f3.at[pl.ds(0, CF)], outd.at[pl.ds(base, CF)])
        pltpu.sync_copy(f4.at[pl.ds(0, CF)], outs.at[pl.ds(base, CF)])


def kernel(md_1_for, means2D):
    mdp = jnp.pad(md_1_for, ((0, NP - N), (0, 0)))
    mep = jnp.pad(means2D, ((0, NP - N), (0, 0)))
    cols = [mdp[:, i].reshape(ROWS, 128) for i in (0, 1, 2, 3, 4)]
    mx = mep[:, 0].reshape(ROWS, 128)
    my = mep[:, 1].reshape(ROWS, 128)
    o0, o1, o2, o3, o4, idx = _pre(cols[0], cols[1], cols[2], cols[3],
                                   cols[4], mx, my)
    r1 = lambda a: a.reshape(NP)
    outc, outd, outs = _make_sc()(r1(o0), r1(o1), r1(o2), r1(o3), r1(o4),
                                  r1(idx))
    return (outc.reshape(H, W, 3),
            outd.reshape(H, W, 1),
            outs.reshape(H, W, 1))


# speedup vs baseline: 15.5489x; 3.3112x over previous
import functools

import jax
import jax.numpy as jnp
from jax import lax
from jax.experimental import pallas as pl
from jax.experimental.pallas import tpu as pltpu
from jax.experimental.pallas import tpu_sc as plsc

H, W = 360, 480
NPIX = H * W
N = 100000
NP = 102400
ROWS = NP // 128
TROWS = ROWS // 16
CB = 10
CPTS = CB * 128
NCHUNK = TROWS // CB
PIX_W = NPIX // 32
PIX_C = NPIX // 16
ZLEN = 2160


def _pre_body(c0, c1, c2, op, dp, mx, my, o0, o1, o2, o3, o4, idx_ref):
    w = 1.0 / (1.0 + jnp.exp(-op[...]))
    x = dp[...]
    d = jnp.maximum(x, 0.0) + jnp.log(1.0 + jnp.exp(-jnp.abs(x)))
    rid = lax.broadcasted_iota(jnp.int32, (ROWS, 128), 0)
    cid = lax.broadcasted_iota(jnp.int32, (ROWS, 128), 1)
    valid = (rid * 128 + cid) < N
    w = jnp.where(valid, w, 0.0)
    px = jnp.clip(jnp.floor(mx[...] * W), 0.0, W - 1)
    py = jnp.clip(jnp.floor(my[...] * H), 0.0, H - 1)
    flat = (py * W + px).astype(jnp.int32)
    idx_ref[...] = jnp.where(valid, flat, rid * 128 + cid)
    o0[...] = c0[...] * w
    o1[...] = c1[...] * w
    o2[...] = c2[...] * w
    o3[...] = d * w
    o4[...] = w


_pre = pl.pallas_call(
    _pre_body,
    out_shape=(
        [jax.ShapeDtypeStruct((ROWS, 128), jnp.float32) for _ in range(5)]
        + [jax.ShapeDtypeStruct((ROWS, 128), jnp.int32)]
    ),
)


@functools.cache
def _make_sc():
  return pl.kernel(
    _sc_body,
    mesh=plsc.VectorSubcoreMesh(core_axis_name="c", subcore_axis_name="s",
                                num_cores=2, num_subcores=16),
    out_type=[jax.ShapeDtypeStruct((NPIX,), jnp.float32) for _ in range(5)],
    scratch_types=[
        pltpu.VMEM((CPTS,), jnp.int32),
        [pltpu.VMEM((CPTS,), jnp.float32) for _ in range(5)],
        [pltpu.VMEM((CPTS,), jnp.float32) for _ in range(5)],
        pltpu.VMEM((ZLEN,), jnp.float32),
        pltpu.VMEM((PIX_W,), jnp.float32),
        [pltpu.VMEM_SHARED((NPIX,), jnp.float32) for _ in range(5)],
        pltpu.SemaphoreType.DMA,
        pltpu.SemaphoreType.DMA,
    ],
    compiler_params=pltpu.CompilerParams(needs_layout_passes=False),
  )


def _sc_body(v0h, v1h, v2h, v3h, v4h, idx_hbm, o0, o1, o2, o3, o4,
             idxv, vbufA, vbufB, zbuf, cbuf, accs, sem_s, sem_l):
    cc = lax.axis_index("c")
    ss = lax.axis_index("s")
    wid = cc * 16 + ss
    vhbm = (v0h, v1h, v2h, v3h, v4h)
    outs = (o0, o1, o2, o3, o4)
    vbufs = (vbufA, vbufB)
    scope = jax.named_scope

    with scope("zero_phase"):
        def _zb(i, carry):
            zbuf[pl.ds(i * 16, 16)] = jnp.zeros((16,), jnp.float32)
            return carry
        lax.fori_loop(0, ZLEN // 16, _zb, 0)
        zcopies = []
        for a in accs:
            for k in range(PIX_C // ZLEN):
                zcopies.append(pltpu.async_copy(
                    zbuf, a.at[pl.ds(ss * PIX_C + k * ZLEN, ZLEN)], sem_l))
        for cp in zcopies:
            cp.wait()
        plsc.subcore_barrier()

    base_pt = ss * TROWS * 128

    def _load(c, buf, sem):
        return [pltpu.async_copy(
            vhbm[ch].at[pl.ds(base_pt + c * CPTS, CPTS)], buf[ch], sem)
            for ch in range(5)]

    with scope("scatter_phase"):
        pltpu.sync_copy(idx_hbm.at[pl.ds(base_pt, CPTS)], idxv)
        for cp in _load(0, vbufs[0], sem_l):
            cp.wait()
        for c in range(NCHUNK):
            cur = vbufs[c % 2]
            streams = [pltpu.async_copy(cur[ch], accs[ch].at[idxv], sem_s,
                                        add=True) for ch in range(5)]
            loads = _load(c + 1, vbufs[(c + 1) % 2], sem_l) \
                if c + 1 < NCHUNK else []
            for cp in loads:
                cp.wait()
            for cp in streams:
                cp.wait()
            if c + 1 < NCHUNK:
                pltpu.sync_copy(
                    idx_hbm.at[pl.ds(base_pt + (c + 1) * CPTS, CPTS)], idxv)
        plsc.subcore_barrier()

    with scope("copyout_phase"):
        base = wid * PIX_W
        for ch in range(5):
            pltpu.sync_copy(accs[ch].at[pl.ds(base, PIX_W)], cbuf)
            pltpu.sync_copy(cbuf, outs[ch].at[pl.ds(base, PIX_W)])


def kernel(md_1_for, means2D):
    mdp = jnp.pad(md_1_for, ((0, NP - N), (0, 0)))
    mep = jnp.pad(means2D, ((0, NP - N), (0, 0)))
    cols = [mdp[:, i].reshape(ROWS, 128) for i in (0, 1, 2, 3, 4)]
    mx = mep[:, 0].reshape(ROWS, 128)
    my = mep[:, 1].reshape(ROWS, 128)
    o0, o1, o2, o3, o4, idx = _pre(cols[0], cols[1], cols[2], cols[3],
                                   cols[4], mx, my)
    r1 = lambda a: a.reshape(NP)
    c0a, c1a, c2a, dpa, wa = _make_sc()(r1(o0), r1(o1), r1(o2), r1(o3),
                                        r1(o4), r1(idx))
    denom = jnp.maximum(wa, 1e-8)
    color = (jnp.stack([c0a, c1a, c2a], axis=-1)
             / denom[:, None]).reshape(H, W, 3)
    depth = (dpa / denom).reshape(H, W, 1)
    sil = jnp.clip(wa, 0.0, 1.0).reshape(H, W, 1)
    return (color, depth, sil)


# speedup vs baseline: 16.6823x; 1.0729x over previous
import functools

import jax
import jax.numpy as jnp
from jax import lax
from jax.experimental import pallas as pl
from jax.experimental.pallas import tpu as pltpu
from jax.experimental.pallas import tpu_sc as plsc

H, W = 360, 480
NPIX = H * W
N = 100000
NP = 102400
ROWS = NP // 128
TROWS = ROWS // 16
CB = 10
CPTS = CB * 128
NCHUNK = TROWS // CB
PIX_W = NPIX // 32
PIX_C = NPIX // 16
ZLEN = 2160


def _pre_body(c0, c1, c2, op, dp, mx, my, o0, o1, o2, o3, o4, idx_ref):
    w = 1.0 / (1.0 + jnp.exp(-op[...]))
    x = dp[...]
    d = jnp.maximum(x, 0.0) + jnp.log(1.0 + jnp.exp(-jnp.abs(x)))
    rid = lax.broadcasted_iota(jnp.int32, (ROWS, 128), 0)
    cid = lax.broadcasted_iota(jnp.int32, (ROWS, 128), 1)
    valid = (rid * 128 + cid) < N
    w = jnp.where(valid, w, 0.0)
    px = jnp.clip(jnp.floor(mx[...] * W), 0.0, W - 1)
    py = jnp.clip(jnp.floor(my[...] * H), 0.0, H - 1)
    flat = (py * W + px).astype(jnp.int32)
    idx_ref[...] = jnp.where(valid, flat, rid * 128 + cid)
    o0[...] = c0[...] * w
    o1[...] = c1[...] * w
    o2[...] = c2[...] * w
    o3[...] = d * w
    o4[...] = w


_pre = pl.pallas_call(
    _pre_body,
    out_shape=(
        [jax.ShapeDtypeStruct((ROWS, 128), jnp.float32) for _ in range(5)]
        + [jax.ShapeDtypeStruct((ROWS, 128), jnp.int32)]
    ),
)


@functools.cache
def _make_sc():
  return pl.kernel(
    _sc_body,
    mesh=plsc.VectorSubcoreMesh(core_axis_name="c", subcore_axis_name="s",
                                num_cores=2, num_subcores=16),
    out_type=[jax.ShapeDtypeStruct((NPIX,), jnp.float32) for _ in range(5)],
    scratch_types=[
        pltpu.VMEM((CPTS,), jnp.int32),
        [pltpu.VMEM((CPTS,), jnp.float32) for _ in range(5)],
        [pltpu.VMEM((CPTS,), jnp.float32) for _ in range(5)],
        pltpu.VMEM((ZLEN,), jnp.float32),
        pltpu.VMEM((PIX_W,), jnp.float32),
        [pltpu.VMEM_SHARED((NPIX,), jnp.float32) for _ in range(5)],
        pltpu.SemaphoreType.DMA,
        pltpu.SemaphoreType.DMA,
    ],
    compiler_params=pltpu.CompilerParams(needs_layout_passes=False),
  )


def _sc_body(v0h, v1h, v2h, v3h, v4h, idx_hbm, o0, o1, o2, o3, o4,
             idxv, vbufA, vbufB, zbuf, cbuf, accs, sem_s, sem_l):
    cc = lax.axis_index("c")
    ss = lax.axis_index("s")
    wid = cc * 16 + ss
    vhbm = (v0h, v1h, v2h, v3h, v4h)
    outs = (o0, o1, o2, o3, o4)
    vbufs = (vbufA, vbufB)
    scope = jax.named_scope

    base_pt = ss * TROWS * 128

    def _for_core(fn3, fn2):
        @pl.when(cc == 0)
        def _():
            fn3()
        @pl.when(cc != 0)
        def _():
            fn2()

    def _load(c, buf, sem, chans):
        return [pltpu.async_copy(
            vhbm[ch].at[pl.ds(base_pt + c * CPTS, CPTS)], buf[ch], sem)
            for ch in chans]

    pf_idx = pltpu.async_copy(idx_hbm.at[pl.ds(base_pt, CPTS)], idxv, sem_s)

    with scope("zero_phase"):
        def _zb(i, carry):
            zbuf[pl.ds(i * 16, 16)] = jnp.zeros((16,), jnp.float32)
            return carry
        lax.fori_loop(0, ZLEN // 16, _zb, 0)
        zcopies = []
        for a in accs:
            for k in range(PIX_C // ZLEN):
                zcopies.append(pltpu.async_copy(
                    zbuf, a.at[pl.ds(ss * PIX_C + k * ZLEN, ZLEN)], sem_l))
        for cp in zcopies:
            cp.wait()
        pf_idx.wait()
        plsc.subcore_barrier()

    with scope("scatter_phase"):
        for c in range(NCHUNK):
            cur = vbufs[c % 2]
            if c == 0:
                l3 = lambda: [cp.wait() for cp in
                              _load(0, vbufs[0], sem_l, (0, 1, 2))]
                l2 = lambda: [cp.wait() for cp in
                              _load(0, vbufs[0], sem_l, (3, 4))]
                _for_core(l3, l2)

            def _scat(chs, cur=cur):
                streams = [pltpu.async_copy(cur[ch], accs[ch].at[idxv],
                                            sem_s, add=True) for ch in chs]
                if c + 1 < NCHUNK:
                    loads = _load(c + 1, vbufs[(c + 1) % 2], sem_l, chs)
                    for cp in loads:
                        cp.wait()
                for cp in streams:
                    cp.wait()
            _for_core(lambda: _scat((0, 1, 2)), lambda: _scat((3, 4)))
            if c + 1 < NCHUNK:
                pltpu.sync_copy(
                    idx_hbm.at[pl.ds(base_pt + (c + 1) * CPTS, CPTS)], idxv)
        plsc.subcore_barrier()

    with scope("copyout_phase"):
        def _cpout(chs):
            for ch in chs:
                for k in range(2):
                    base = ss * PIX_C + k * PIX_W
                    pltpu.sync_copy(accs[ch].at[pl.ds(base, PIX_W)], cbuf)
                    pltpu.sync_copy(cbuf, outs[ch].at[pl.ds(base, PIX_W)])
        _for_core(lambda: _cpout((0, 1, 2)), lambda: _cpout((3, 4)))


def kernel(md_1_for, means2D):
    mdp = jnp.pad(md_1_for, ((0, NP - N), (0, 0)))
    mep = jnp.pad(means2D, ((0, NP - N), (0, 0)))
    cols = [mdp[:, i].reshape(ROWS, 128) for i in (0, 1, 2, 3, 4)]
    mx = mep[:, 0].reshape(ROWS, 128)
    my = mep[:, 1].reshape(ROWS, 128)
    o0, o1, o2, o3, o4, idx = _pre(cols[0], cols[1], cols[2], cols[3],
                                   cols[4], mx, my)
    r1 = lambda a: a.reshape(NP)
    c0a, c1a, c2a, dpa, wa = _make_sc()(r1(o0), r1(o1), r1(o2), r1(o3),
                                        r1(o4), r1(idx))
    denom = jnp.maximum(wa, 1e-8)
    color = (jnp.stack([c0a, c1a, c2a], axis=-1)
             / denom[:, None]).reshape(H, W, 3)
    depth = (dpa / denom).reshape(H, W, 1)
    sil = jnp.clip(wa, 0.0, 1.0).reshape(H, W, 1)
    return (color, depth, sil)
